# baseline (speedup 1.0000x reference)
import jax
import jax.numpy as jnp
from jax.experimental import pallas as pl

NUM_CLASSES = 1000


def _body(x_ref, o_ref):
    cls = jax.lax.broadcasted_iota(jnp.int32, o_ref.shape, 1)
    o_ref[...] = (cls == x_ref[...][:, None, :]).astype(jnp.float32)


def kernel(x):
    B, L = x.shape
    BB = 8
    return pl.pallas_call(
        _body,
        grid=(B // BB,),
        in_specs=[pl.BlockSpec((BB, L), lambda i: (i, 0))],
        out_specs=pl.BlockSpec((BB, NUM_CLASSES, L), lambda i: (i, 0, 0)),
        out_shape=jax.ShapeDtypeStruct((B, NUM_CLASSES, L), jnp.float32),
    )(x)


# speedup vs baseline: 8.2982x; 8.2982x over previous
import jax
import jax.numpy as jnp
from jax.experimental import pallas as pl

NUM_CLASSES = 1000


def _body(xt_ref, o_ref):
    cls = jax.lax.broadcasted_iota(jnp.int32, o_ref.shape, 1)
    o_ref[...] = (cls == xt_ref[...]).astype(jnp.float32)


def kernel(x):
    B, L = x.shape
    xt = jnp.swapaxes(x, 0, 1).reshape(L, 1, B)
    p = pl.pallas_call(
        _body,
        grid=(L,),
        in_specs=[pl.BlockSpec((1, 1, B), lambda i: (i, 0, 0))],
        out_specs=pl.BlockSpec((1, NUM_CLASSES, B), lambda i: (i, 0, 0)),
        out_shape=jax.ShapeDtypeStruct((L, NUM_CLASSES, B), jnp.float32),
    )(xt)
    return jnp.transpose(p, (2, 1, 0))
